# baseline (speedup 1.0000x reference)
import functools

import jax
import jax.numpy as jnp
from jax import lax
from jax.experimental import pallas as pl
from jax.experimental.pallas import tpu as pltpu
from jax.experimental.pallas import tpu_sc as plsc

N = 10000
E = 160000
C = 256
HALF = 128
NC = 2
NS = 16
B = 128
NB = 80
NBC = 40
EPAD = NS * NB * B
NPAD = 10112
ZROWS = 632
OROWS = 624

_f32 = jnp.float32
_MESH = dict(core_axis_name="c", subcore_axis_name="s",
             num_cores=NC, num_subcores=NS)


def _sc_agg_kernel():
    @functools.partial(
        pl.kernel,
        out_type=jax.ShapeDtypeStruct((NC, N, HALF), _f32),
        mesh=plsc.VectorSubcoreMesh(**_MESH),
        scratch_types=[
            pltpu.VMEM((NB // 2, B), jnp.int32),
            pltpu.VMEM((NB // 2, B), jnp.int32),
            pltpu.VMEM((B, HALF), _f32),
            pltpu.VMEM((B, HALF), _f32),
            pltpu.VMEM_SHARED((NPAD, HALF), _f32),
            pltpu.SemaphoreType.DMA,
            pltpu.SemaphoreType.DMA,
        ],
    )
    def sc_agg(x2_hbm, idx2_hbm, dstp_hbm, agg_out,
               sidx, didx, r0, r1, agg_sh, g0, g1):
        c = lax.axis_index("c")
        s = lax.axis_index("s")
        rows_l = (r0, r1)
        gsems = (g0, g1)
        NBUF = 2
        NBH = NB // 2

        def zrows_body(i, _):
            for j in range(HALF // 16):
                r0[i, pl.ds(j * 16, 16)] = jnp.zeros((16,), _f32)
            return 0
        lax.fori_loop(0, B, zrows_body, 0)

        zbase = s * ZROWS
        for r, sz in ((0, 128), (128, 128), (256, 128), (384, 128),
                      (512, ZROWS - 512)):
            pltpu.sync_copy(r0.at[pl.ds(0, sz)],
                            agg_sh.at[pl.ds(zbase + r, sz)])

        plsc.subcore_barrier()

        for h in range(2):
            pltpu.sync_copy(
                idx2_hbm.at[c].at[pl.ds(s * NB + h * NBH, NBH)], sidx)
            pltpu.sync_copy(
                dstp_hbm.at[pl.ds(s * NB + h * NBH, NBH)], didx)

            for k in range(NBUF):
                pltpu.async_copy(
                    x2_hbm.at[sidx.at[k]], rows_l[k], gsems[k])

            def qbody(q, _):
                base = q * NBUF
                scat = []
                for k in range(NBUF):
                    pltpu.make_async_copy(
                        x2_hbm.at[pl.ds(0, B)], rows_l[k],
                        gsems[k]).wait()
                    scat.append(pltpu.async_copy(
                        rows_l[k], agg_sh.at[didx.at[base + k]],
                        gsems[k], add=True))
                for k in range(NBUF):
                    scat[k].wait()
                    nxt = base + NBUF + k

                    @pl.when(nxt < NBH)
                    def _refill():
                        pltpu.async_copy(
                            x2_hbm.at[sidx.at[nxt]], rows_l[k],
                            gsems[k])
                return 0
            lax.fori_loop(0, NBH // NBUF, qbody, 0)

        plsc.subcore_barrier()

        obase = s * OROWS
        pltpu.sync_copy(agg_sh.at[pl.ds(obase, OROWS)],
                        agg_out.at[c].at[pl.ds(obase, OROWS)])

        @pl.when(s == NS - 1)
        def _tail():
            tbase = NS * OROWS
            pltpu.sync_copy(agg_sh.at[pl.ds(tbase, N - tbase)],
                            agg_out.at[c].at[pl.ds(tbase, N - tbase)])

    return sc_agg


def _sc_cnt_kernel():
    @functools.partial(
        pl.kernel,
        out_type=jax.ShapeDtypeStruct((NC, N, HALF), _f32),
        mesh=plsc.VectorSubcoreMesh(**_MESH),
        scratch_types=[
            pltpu.VMEM((NBC, B), jnp.int32),
            pltpu.VMEM((B, HALF), _f32),
            pltpu.VMEM_SHARED((NPAD, HALF), _f32),
            pltpu.SemaphoreType.DMA,
        ],
    )
    def sc_cnt(dstp_hbm, cnt_out, didx, ones_v, cnt_sh, sem):
        c = lax.axis_index("c")
        s = lax.axis_index("s")

        def fill_body_zero(i, _):
            for j in range(HALF // 16):
                ones_v[i, pl.ds(j * 16, 16)] = jnp.zeros((16,), _f32)
            return 0
        lax.fori_loop(0, B, fill_body_zero, 0)

        zbase = s * ZROWS
        for r, sz in ((0, 128), (128, 128), (256, 128), (384, 128),
                      (512, ZROWS - 512)):
            pltpu.sync_copy(ones_v.at[pl.ds(0, sz)],
                            cnt_sh.at[pl.ds(zbase + r, sz)])

        def fill_body_one(i, _):
            for j in range(HALF // 16):
                ones_v[i, pl.ds(j * 16, 16)] = jnp.ones((16,), _f32)
            return 0
        lax.fori_loop(0, B, fill_body_one, 0)

        pltpu.sync_copy(
            dstp_hbm.at[pl.ds((c * NS + s) * NBC, NBC)], didx)

        plsc.subcore_barrier()

        def ebody(b, _):
            pltpu.sync_copy(ones_v, cnt_sh.at[didx.at[b]], add=True)
            return 0
        lax.fori_loop(0, NBC, ebody, 0)

        plsc.subcore_barrier()

        obase = s * OROWS
        pltpu.sync_copy(cnt_sh.at[pl.ds(obase, OROWS)],
                        cnt_out.at[c].at[pl.ds(obase, OROWS)])

        @pl.when(s == NS - 1)
        def _tail():
            tbase = NS * OROWS
            pltpu.sync_copy(cnt_sh.at[pl.ds(tbase, N - tbase)],
                            cnt_out.at[c].at[pl.ds(tbase, N - tbase)])

    return sc_cnt


def _tc_layer(agg, cnt, x_il, wlt, wrt, b, *, relu, final):
    blk = 1000
    grid = (N // blk,)

    def body(a_r, cnt_r, x_r, wl_r, wr_r, b_r, o_r):
        deg = cnt_r[0, :, 0:1] + cnt_r[1, :, 0:1]
        rc = 1.0 / jnp.maximum(deg, 1.0)
        acc = jnp.dot(a_r[0] * rc, wl_r[0:HALF, :],
                      preferred_element_type=_f32)
        acc += jnp.dot(a_r[1] * rc, wl_r[HALF:C, :],
                       preferred_element_type=_f32)
        acc += jnp.dot(x_r[:, 0, :], wr_r[0:HALF, :],
                       preferred_element_type=_f32)
        acc += jnp.dot(x_r[:, 1, :], wr_r[HALF:C, :],
                       preferred_element_type=_f32)
        acc += b_r[...]
        if relu:
            acc = jnp.maximum(acc, 0.0)
        if final:
            o_r[...] = acc
        else:
            o_r[:, 0, :] = acc[:, 0:HALF]
            o_r[:, 1, :] = acc[:, HALF:C]

    in_specs = [
        pl.BlockSpec((NC, blk, HALF), lambda i: (0, i, 0)),
        pl.BlockSpec((NC, blk, HALF), lambda i: (0, i, 0)),
        pl.BlockSpec((blk, NC, HALF), lambda i: (i, 0, 0)),
        pl.BlockSpec((C, C), lambda i: (0, 0)),
        pl.BlockSpec((C, C), lambda i: (0, 0)),
        pl.BlockSpec((1, C), lambda i: (0, 0)),
    ]
    if final:
        out_spec = pl.BlockSpec((blk, C), lambda i: (i, 0))
        out_shape = jax.ShapeDtypeStruct((N, C), _f32)
    else:
        out_spec = pl.BlockSpec((blk, NC, HALF), lambda i: (i, 0, 0))
        out_shape = jax.ShapeDtypeStruct((N, NC, HALF), _f32)

    return pl.pallas_call(
        body, grid=grid, in_specs=in_specs, out_specs=out_spec,
        out_shape=out_shape,
    )(agg, cnt, x_il, wlt, wrt, b)


def kernel(x, edge_index, W1l, b1l, W1r, W2l, b2l, W2r):
    ei = edge_index.astype(jnp.int32)
    perm = jnp.argsort(ei[0])
    src, dst = ei[0][perm], ei[1][perm]
    pad = EPAD - E
    src_p = jnp.concatenate([src, jnp.zeros((pad,), jnp.int32)])
    idx2 = jnp.stack([src_p * 2, src_p * 2 + 1]).reshape(NC, NS * NB, B)
    dstp = jnp.concatenate(
        [dst, jnp.full((pad,), N, jnp.int32)]).reshape(NS * NB, B)

    x_il = x.reshape(N, NC, HALF)
    x2 = x.reshape(NC * N, HALF)

    sc_agg = _sc_agg_kernel()
    sc_cnt = _sc_cnt_kernel()

    cnt = sc_cnt(dstp)
    agg1 = sc_agg(x2, idx2, dstp)

    h_il = _tc_layer(agg1, cnt, x_il,
                     W1l.T, W1r.T, b1l.reshape(1, C),
                     relu=True, final=False)

    h2 = h_il.reshape(NC * N, HALF)
    agg2 = sc_agg(h2, idx2, dstp)

    out = _tc_layer(agg2, cnt, h_il,
                    W2l.T, W2r.T, b2l.reshape(1, C),
                    relu=False, final=True)
    return out


# speedup vs baseline: 1.4775x; 1.4775x over previous
import functools

import jax
import jax.numpy as jnp
from jax import lax
from jax.experimental import pallas as pl
from jax.experimental.pallas import tpu as pltpu
from jax.experimental.pallas import tpu_sc as plsc

N = 10000
E = 160000
C = 256
HALF = 128
NC = 2
NS = 16
B = 64
NB = 160
NBC = 80
EPAD = NS * NB * B
NPAD = 10112
ZROWS = 632
OROWS = 624

_f32 = jnp.float32
_MESH = dict(core_axis_name="c", subcore_axis_name="s",
             num_cores=NC, num_subcores=NS)


def _sc_agg_kernel():
    @functools.partial(
        pl.kernel,
        out_type=jax.ShapeDtypeStruct((NC, N, HALF), _f32),
        mesh=plsc.VectorSubcoreMesh(**_MESH),
        scratch_types=[
            pltpu.VMEM((NB // 4, B), jnp.int32),
            pltpu.VMEM((NB // 4, B), jnp.int32),
            pltpu.VMEM((B, HALF), _f32),
            pltpu.VMEM((B, HALF), _f32),
            pltpu.VMEM((B, HALF), _f32),
            pltpu.VMEM((B, HALF), _f32),
            pltpu.VMEM_SHARED((NPAD, HALF), _f32),
            pltpu.SemaphoreType.DMA,
            pltpu.SemaphoreType.DMA,
            pltpu.SemaphoreType.DMA,
            pltpu.SemaphoreType.DMA,
        ],
    )
    def sc_agg(x2_hbm, idx2_hbm, dstp_hbm, agg_out,
               sidx, didx, r0, r1, r2, r3, agg_sh, g0, g1, g2, g3):
        c = lax.axis_index("c")
        s = lax.axis_index("s")
        rows_l = (r0, r1, r2, r3)
        gsems = (g0, g1, g2, g3)
        NBUF = 4
        NBH = NB // 4

        def zrows_body(i, _):
            for j in range(HALF // 16):
                r0[i, pl.ds(j * 16, 16)] = jnp.zeros((16,), _f32)
            return 0
        lax.fori_loop(0, B, zrows_body, 0)

        zbase = s * ZROWS
        for r in range(0, 576, 64):
            pltpu.sync_copy(r0, agg_sh.at[pl.ds(zbase + r, 64)])
        pltpu.sync_copy(r0.at[pl.ds(0, ZROWS - 576)],
                        agg_sh.at[pl.ds(zbase + 576, ZROWS - 576)])

        plsc.subcore_barrier()

        for h in range(4):
            pltpu.sync_copy(
                idx2_hbm.at[c].at[pl.ds(s * NB + h * NBH, NBH)], sidx)
            pltpu.sync_copy(
                dstp_hbm.at[pl.ds(s * NB + h * NBH, NBH)], didx)

            for k in range(NBUF):
                pltpu.async_copy(
                    x2_hbm.at[sidx.at[k]], rows_l[k], gsems[k])

            def qbody(q, _):
                base = q * NBUF
                scat = []
                for k in range(NBUF):
                    pltpu.make_async_copy(
                        x2_hbm.at[pl.ds(0, B)], rows_l[k],
                        gsems[k]).wait()
                    scat.append(pltpu.async_copy(
                        rows_l[k], agg_sh.at[didx.at[base + k]],
                        gsems[k], add=True))
                for k in range(NBUF):
                    scat[k].wait()
                    nxt = base + NBUF + k

                    @pl.when(nxt < NBH)
                    def _refill():
                        pltpu.async_copy(
                            x2_hbm.at[sidx.at[nxt]], rows_l[k],
                            gsems[k])
                return 0
            lax.fori_loop(0, NBH // NBUF, qbody, 0)

        plsc.subcore_barrier()

        obase = s * OROWS
        pltpu.sync_copy(agg_sh.at[pl.ds(obase, OROWS)],
                        agg_out.at[c].at[pl.ds(obase, OROWS)])

        @pl.when(s == NS - 1)
        def _tail():
            tbase = NS * OROWS
            pltpu.sync_copy(agg_sh.at[pl.ds(tbase, N - tbase)],
                            agg_out.at[c].at[pl.ds(tbase, N - tbase)])

    return sc_agg


def _sc_cnt_kernel():
    @functools.partial(
        pl.kernel,
        out_type=jax.ShapeDtypeStruct((NC, N, HALF), _f32),
        mesh=plsc.VectorSubcoreMesh(**_MESH),
        scratch_types=[
            pltpu.VMEM((NBC, B), jnp.int32),
            pltpu.VMEM((B, HALF), _f32),
            pltpu.VMEM_SHARED((NPAD, HALF), _f32),
            pltpu.SemaphoreType.DMA,
        ],
    )
    def sc_cnt(dstp_hbm, cnt_out, didx, ones_v, cnt_sh, sem):
        c = lax.axis_index("c")
        s = lax.axis_index("s")

        def fill_body_zero(i, _):
            for j in range(HALF // 16):
                ones_v[i, pl.ds(j * 16, 16)] = jnp.zeros((16,), _f32)
            return 0
        lax.fori_loop(0, B, fill_body_zero, 0)

        zbase = s * ZROWS
        for r in range(0, 576, 64):
            pltpu.sync_copy(ones_v, cnt_sh.at[pl.ds(zbase + r, 64)])
        pltpu.sync_copy(ones_v.at[pl.ds(0, ZROWS - 576)],
                        cnt_sh.at[pl.ds(zbase + 576, ZROWS - 576)])

        def fill_body_one(i, _):
            for j in range(HALF // 16):
                ones_v[i, pl.ds(j * 16, 16)] = jnp.ones((16,), _f32)
            return 0
        lax.fori_loop(0, B, fill_body_one, 0)

        pltpu.sync_copy(
            dstp_hbm.at[pl.ds((c * NS + s) * NBC, NBC)], didx)

        plsc.subcore_barrier()

        def ebody(b, _):
            pltpu.sync_copy(ones_v, cnt_sh.at[didx.at[b]], add=True)
            return 0
        lax.fori_loop(0, NBC, ebody, 0)

        plsc.subcore_barrier()

        obase = s * OROWS
        pltpu.sync_copy(cnt_sh.at[pl.ds(obase, OROWS)],
                        cnt_out.at[c].at[pl.ds(obase, OROWS)])

        @pl.when(s == NS - 1)
        def _tail():
            tbase = NS * OROWS
            pltpu.sync_copy(cnt_sh.at[pl.ds(tbase, N - tbase)],
                            cnt_out.at[c].at[pl.ds(tbase, N - tbase)])

    return sc_cnt


def _tc_layer(agg, cnt, x_il, wlt, wrt, b, *, relu, final):
    blk = 1000
    grid = (N // blk,)

    def body(a_r, cnt_r, x_r, wl_r, wr_r, b_r, o_r):
        deg = cnt_r[0, :, 0:1] + cnt_r[1, :, 0:1]
        rc = 1.0 / jnp.maximum(deg, 1.0)
        acc = jnp.dot(a_r[0] * rc, wl_r[0:HALF, :],
                      preferred_element_type=_f32)
        acc += jnp.dot(a_r[1] * rc, wl_r[HALF:C, :],
                       preferred_element_type=_f32)
        acc += jnp.dot(x_r[:, 0, :], wr_r[0:HALF, :],
                       preferred_element_type=_f32)
        acc += jnp.dot(x_r[:, 1, :], wr_r[HALF:C, :],
                       preferred_element_type=_f32)
        acc += b_r[...]
        if relu:
            acc = jnp.maximum(acc, 0.0)
        if final:
            o_r[...] = acc
        else:
            o_r[:, 0, :] = acc[:, 0:HALF]
            o_r[:, 1, :] = acc[:, HALF:C]

    in_specs = [
        pl.BlockSpec((NC, blk, HALF), lambda i: (0, i, 0)),
        pl.BlockSpec((NC, blk, HALF), lambda i: (0, i, 0)),
        pl.BlockSpec((blk, NC, HALF), lambda i: (i, 0, 0)),
        pl.BlockSpec((C, C), lambda i: (0, 0)),
        pl.BlockSpec((C, C), lambda i: (0, 0)),
        pl.BlockSpec((1, C), lambda i: (0, 0)),
    ]
    if final:
        out_spec = pl.BlockSpec((blk, C), lambda i: (i, 0))
        out_shape = jax.ShapeDtypeStruct((N, C), _f32)
    else:
        out_spec = pl.BlockSpec((blk, NC, HALF), lambda i: (i, 0, 0))
        out_shape = jax.ShapeDtypeStruct((N, NC, HALF), _f32)

    return pl.pallas_call(
        body, grid=grid, in_specs=in_specs, out_specs=out_spec,
        out_shape=out_shape,
    )(agg, cnt, x_il, wlt, wrt, b)


def kernel(x, edge_index, W1l, b1l, W1r, W2l, b2l, W2r):
    ei = edge_index.astype(jnp.int32)
    src, dst = ei[0], ei[1]
    pad = EPAD - E
    src_p = jnp.concatenate([src, jnp.zeros((pad,), jnp.int32)])
    idx2 = jnp.stack([src_p * 2, src_p * 2 + 1]).reshape(NC, NS * NB, B)
    dstp = jnp.concatenate(
        [dst, jnp.full((pad,), N, jnp.int32)]).reshape(NS * NB, B)

    x_il = x.reshape(N, NC, HALF)
    x2 = x.reshape(NC * N, HALF)

    sc_agg = _sc_agg_kernel()
    sc_cnt = _sc_cnt_kernel()

    cnt = sc_cnt(dstp)
    agg1 = sc_agg(x2, idx2, dstp)

    h_il = _tc_layer(agg1, cnt, x_il,
                     W1l.T, W1r.T, b1l.reshape(1, C),
                     relu=True, final=False)

    h2 = h_il.reshape(NC * N, HALF)
    agg2 = sc_agg(h2, idx2, dstp)

    out = _tc_layer(agg2, cnt, h_il,
                    W2l.T, W2r.T, b2l.reshape(1, C),
                    relu=False, final=True)
    return out
